# Initial kernel scaffold; baseline (speedup 1.0000x reference)
import jax
import jax.numpy as jnp
from jax.experimental import pallas as pl


def kernel(organs_idx, cells_idx, subcells_idx, W_o, b_o, g_o, be_o, W_c, b_c, g_c, be_c, W_s, b_s, g_s, be_s):
    raise NotImplementedError("write your pallas kernel here")


# speedup vs baseline: 29.5943x; 29.5943x over previous
import functools

import jax
import jax.numpy as jnp
from jax import lax
from jax.experimental import pallas as pl
from jax.experimental.pallas import tpu as pltpu
from jax.experimental.pallas import tpu_sc as plsc

B = 4096
L = 4
V = 8192
D = 512
EPS = 1e-5

NW = 32
PW = B // NW
CS = 32
NCHUNK = PW // CS
ROWS = CS * L

_mesh = plsc.VectorSubcoreMesh(core_axis_name="c", subcore_axis_name="s")


@functools.partial(
    pl.kernel,
    mesh=_mesh,
    out_type=jax.ShapeDtypeStruct((3, B, D), jnp.float32),
    scratch_types=[
        pltpu.VMEM((3, L, PW), jnp.int32),
        pltpu.VMEM((ROWS,), jnp.int32),
        pltpu.VMEM((ROWS, D), jnp.float32),
        pltpu.VMEM((CS, D), jnp.float32),
        pltpu.SemaphoreType.DMA,
    ],
)
def _sc_gather_sum(idx_hbm, to_hbm, tc_hbm, ts_hbm, out_hbm,
                   idx_v, gi_v, rows_v, sums_v, sem):
    wid = lax.axis_index("s") * 2 + lax.axis_index("c")
    base = wid * PW
    for t in range(3):
        for l in range(L):
            pltpu.sync_copy(idx_hbm.at[t, l, pl.ds(base, PW)], idx_v.at[t, l])
    for t, tbl in enumerate((to_hbm, tc_hbm, ts_hbm)):
        for c in range(NCHUNK):
            for g in range(CS // 16):
                s0 = c * CS + g * 16
                i0 = idx_v[t, 0, pl.ds(s0, 16)]
                i1 = idx_v[t, 1, pl.ds(s0, 16)]
                i2 = idx_v[t, 2, pl.ds(s0, 16)]
                i3 = idx_v[t, 3, pl.ds(s0, 16)]
                d1 = jnp.where(i1 != i0, i1, V)
                d2 = jnp.where((i2 != i0) & (i2 != i1), i2, V)
                d3 = jnp.where((i3 != i0) & (i3 != i1) & (i3 != i2), i3, V)
                gi_v[pl.ds(g * 16, 16)] = i0
                gi_v[pl.ds(CS + g * 16, 16)] = d1
                gi_v[pl.ds(2 * CS + g * 16, 16)] = d2
                gi_v[pl.ds(3 * CS + g * 16, 16)] = d3
            pltpu.async_copy(tbl.at[gi_v], rows_v, sem).wait()

            def red_body(s, carry):
                for dblk in range(D // 16):
                    sl = pl.ds(dblk * 16, 16)
                    sums_v[s, sl] = (rows_v[s, sl] + rows_v[CS + s, sl]
                                     + rows_v[2 * CS + s, sl]
                                     + rows_v[3 * CS + s, sl])
                return carry

            lax.fori_loop(0, CS, red_body, 0)
            pltpu.sync_copy(sums_v, out_hbm.at[t, pl.ds(base + c * CS, CS)])


def _tc_ln_body(x_ref, b_ref, g_ref, be_ref, o_ref):
    x = x_ref[...] + b_ref[...][:, None, :]
    m = jnp.mean(x, axis=-1, keepdims=True)
    xc = x - m
    v = jnp.mean(xc * xc, axis=-1, keepdims=True)
    y = xc * lax.rsqrt(v + EPS) * g_ref[...][:, None, :] + be_ref[...][:, None, :]
    o_ref[...] = (y[0] + y[1] + y[2]) * (1.0 / 3.0)


_BB = 512


def _tc_ln(sums, bias, gain, beta):
    return pl.pallas_call(
        _tc_ln_body,
        grid=(B // _BB,),
        in_specs=[
            pl.BlockSpec((3, _BB, D), lambda i: (0, i, 0)),
            pl.BlockSpec((3, D), lambda i: (0, 0)),
            pl.BlockSpec((3, D), lambda i: (0, 0)),
            pl.BlockSpec((3, D), lambda i: (0, 0)),
        ],
        out_specs=pl.BlockSpec((_BB, D), lambda i: (i, 0)),
        out_shape=jax.ShapeDtypeStruct((B, D), jnp.float32),
    )(sums, bias, gain, beta)


def kernel(organs_idx, cells_idx, subcells_idx,
           W_o, b_o, g_o, be_o,
           W_c, b_c, g_c, be_c,
           W_s, b_s, g_s, be_s):
    idx = jnp.stack([organs_idx.T, cells_idx.T, subcells_idx.T])
    pad = jnp.zeros((8, D), jnp.float32)
    tbl_o = jnp.concatenate([W_o.T, pad])
    tbl_c = jnp.concatenate([W_c.T, pad])
    tbl_s = jnp.concatenate([W_s.T, pad])
    sums = _sc_gather_sum(idx, tbl_o, tbl_c, tbl_s)
    bias = jnp.stack([b_o, b_c, b_s])
    gain = jnp.stack([g_o, g_c, g_s])
    beta = jnp.stack([be_o, be_c, be_s])
    return _tc_ln(sums, bias, gain, beta)


# speedup vs baseline: 36.6776x; 1.2393x over previous
import functools

import jax
import jax.numpy as jnp
from jax import lax
from jax.experimental import pallas as pl
from jax.experimental.pallas import tpu as pltpu
from jax.experimental.pallas import tpu_sc as plsc

B = 4096
L = 4
V = 8192
D = 512
EPS = 1e-5

NW = 32
PW = B // NW
CS = 16
NCHUNK = PW // CS
ROWS = CS * L

VP = V + 8
NK = 3 * NCHUNK

_mesh = plsc.VectorSubcoreMesh(core_axis_name="c", subcore_axis_name="s")


@functools.partial(
    pl.kernel,
    mesh=_mesh,
    out_type=jax.ShapeDtypeStruct((3 * B, D), jnp.float32),
    scratch_types=[
        pltpu.VMEM((3, L, PW), jnp.int32),
        pltpu.VMEM((ROWS,), jnp.int32),
        pltpu.VMEM((ROWS,), jnp.int32),
        pltpu.VMEM((ROWS, D), jnp.float32),
        pltpu.VMEM((ROWS, D), jnp.float32),
        pltpu.VMEM((CS, D), jnp.float32),
        pltpu.VMEM((CS, D), jnp.float32),
        pltpu.SemaphoreType.DMA,
        pltpu.SemaphoreType.DMA,
        pltpu.SemaphoreType.DMA,
        pltpu.SemaphoreType.DMA,
    ],
)
def _sc_gather_sum(idx_hbm, tbl_hbm, out_hbm,
                   idx_v, gi0, gi1, rows0, rows1, sums0, sums1,
                   gs0, gs1, ws0, ws1):
    wid = lax.axis_index("s") * 2 + lax.axis_index("c")
    base = wid * PW
    gi = (gi0, gi1)
    rows = (rows0, rows1)
    sums = (sums0, sums1)
    gsem = (gs0, gs1)
    wsem = (ws0, ws1)

    for t in range(3):
        for l in range(L):
            pltpu.sync_copy(idx_hbm.at[t, l, pl.ds(base, PW)], idx_v.at[t, l])

    def start_gather(k, p):
        t = k // NCHUNK
        s0 = (k % NCHUNK) * CS
        off = t * VP
        g = gi[p]
        i0 = idx_v[t, 0, pl.ds(s0, 16)]
        i1 = idx_v[t, 1, pl.ds(s0, 16)]
        i2 = idx_v[t, 2, pl.ds(s0, 16)]
        i3 = idx_v[t, 3, pl.ds(s0, 16)]
        g[pl.ds(0, 16)] = i0 + off
        g[pl.ds(16, 16)] = jnp.where(i1 != i0, i1, V) + off
        g[pl.ds(32, 16)] = jnp.where((i2 != i0) & (i2 != i1), i2, V) + off
        g[pl.ds(48, 16)] = jnp.where((i3 != i0) & (i3 != i1) & (i3 != i2),
                                     i3, V) + off
        pltpu.async_copy(tbl_hbm.at[g], rows[p], gsem[p])

    def wait_gather(p):
        pltpu.make_async_copy(tbl_hbm.at[gi[p]], rows[p], gsem[p]).wait()

    def reduce_chunk(p):
        r, s = rows[p], sums[p]

        def body(i, carry):
            for dblk in range(D // 16):
                sl = pl.ds(dblk * 16, 16)
                s[i, sl] = (r[i, sl] + r[CS + i, sl]
                            + r[2 * CS + i, sl] + r[3 * CS + i, sl])
            return carry

        lax.fori_loop(0, CS, body, 0)

    def start_wb(k, p):
        q = (k // NCHUNK) * B + base + (k % NCHUNK) * CS
        pltpu.async_copy(sums[p], out_hbm.at[pl.ds(q, CS)], wsem[p])

    def wait_wb(p):
        pltpu.make_async_copy(sums[p], out_hbm.at[pl.ds(0, CS)],
                              wsem[p]).wait()

    def chunk(k, p, wait_prev_wb, gather_ahead):
        wait_gather(p)
        if wait_prev_wb:
            wait_wb(p)
        reduce_chunk(p)
        start_wb(k, p)
        if gather_ahead:
            start_gather(k + 2, p)

    start_gather(0, 0)
    start_gather(1, 1)
    chunk(0, 0, False, True)
    chunk(1, 1, False, True)

    def pair(i, carry):
        k = 2 + 2 * i
        chunk(k, 0, True, True)
        chunk(k + 1, 1, True, True)
        return carry

    lax.fori_loop(0, (NK - 4) // 2, pair, 0)
    chunk(NK - 2, 0, True, False)
    chunk(NK - 1, 1, True, False)
    wait_wb(0)
    wait_wb(1)


def _tc_ln_body(x_ref, b_ref, g_ref, be_ref, o_ref):
    x = x_ref[...] + b_ref[...][:, None, :]
    m = jnp.mean(x, axis=-1, keepdims=True)
    xc = x - m
    v = jnp.mean(xc * xc, axis=-1, keepdims=True)
    y = xc * lax.rsqrt(v + EPS) * g_ref[...][:, None, :] + be_ref[...][:, None, :]
    o_ref[...] = (y[0] + y[1] + y[2]) * (1.0 / 3.0)


_BB = 512


def _tc_ln(sums, bias, gain, beta):
    return pl.pallas_call(
        _tc_ln_body,
        grid=(B // _BB,),
        in_specs=[
            pl.BlockSpec((3, _BB, D), lambda i: (0, i, 0)),
            pl.BlockSpec((3, D), lambda i: (0, 0)),
            pl.BlockSpec((3, D), lambda i: (0, 0)),
            pl.BlockSpec((3, D), lambda i: (0, 0)),
        ],
        out_specs=pl.BlockSpec((_BB, D), lambda i: (i, 0)),
        out_shape=jax.ShapeDtypeStruct((B, D), jnp.float32),
    )(sums, bias, gain, beta)


def kernel(organs_idx, cells_idx, subcells_idx,
           W_o, b_o, g_o, be_o,
           W_c, b_c, g_c, be_c,
           W_s, b_s, g_s, be_s):
    idx = jnp.stack([organs_idx.T, cells_idx.T, subcells_idx.T])
    pad = jnp.zeros((8, D), jnp.float32)
    tbl = jnp.concatenate([W_o.T, pad, W_c.T, pad, W_s.T, pad])
    sums = _sc_gather_sum(idx, tbl).reshape(3, B, D)
    bias = jnp.stack([b_o, b_c, b_s])
    gain = jnp.stack([g_o, g_c, g_s])
    beta = jnp.stack([be_o, be_c, be_s])
    return _tc_ln(sums, bias, gain, beta)


# speedup vs baseline: 40.4952x; 1.1041x over previous
import functools

import jax
import jax.numpy as jnp
from jax import lax
from jax.experimental import pallas as pl
from jax.experimental.pallas import tpu as pltpu
from jax.experimental.pallas import tpu_sc as plsc

B = 4096
L = 4
V = 8192
D = 512
EPS = 1e-5

NW = 32
PW = B // NW
CS = 16
NCHUNK = PW // CS
ROWS = CS * L

VPAD = V + 512

_mesh = plsc.VectorSubcoreMesh(core_axis_name="c", subcore_axis_name="s")


def _tc_transpose_body(w_ref, o_ref):
    i = pl.program_id(0)

    @pl.when(i < V // 512)
    def _():
        o_ref[...] = w_ref[...].T

    @pl.when(i == V // 512)
    def _():
        o_ref[...] = jnp.zeros_like(o_ref)


def _tc_transpose(w):
    return pl.pallas_call(
        _tc_transpose_body,
        grid=(V // 512 + 1,),
        in_specs=[pl.BlockSpec((D, 512), lambda i: (0, jnp.minimum(i, V // 512 - 1)))],
        out_specs=pl.BlockSpec((512, D), lambda i: (i, 0)),
        out_shape=jax.ShapeDtypeStruct((VPAD, D), jnp.float32),
    )(w)


@functools.partial(
    pl.kernel,
    mesh=_mesh,
    out_type=jax.ShapeDtypeStruct((B, D), jnp.float32),
    scratch_types=[
        pltpu.VMEM((L, PW), jnp.int32),
        pltpu.VMEM((ROWS,), jnp.int32),
        pltpu.VMEM((ROWS,), jnp.int32),
        pltpu.VMEM((ROWS, D), jnp.float32),
        pltpu.VMEM((ROWS, D), jnp.float32),
        pltpu.VMEM((CS, D), jnp.float32),
        pltpu.VMEM((CS, D), jnp.float32),
        pltpu.SemaphoreType.DMA,
        pltpu.SemaphoreType.DMA,
        pltpu.SemaphoreType.DMA,
        pltpu.SemaphoreType.DMA,
    ],
)
def _sc_gather_sum(idx_hbm, tbl_hbm, out_hbm,
                   idx_v, gi0, gi1, rows0, rows1, sums0, sums1,
                   gs0, gs1, ws0, ws1):
    wid = lax.axis_index("s") * 2 + lax.axis_index("c")
    base = wid * PW
    gi = (gi0, gi1)
    rows = (rows0, rows1)
    sums = (sums0, sums1)
    gsem = (gs0, gs1)
    wsem = (ws0, ws1)

    for l in range(L):
        pltpu.sync_copy(idx_hbm.at[l, pl.ds(base, PW)], idx_v.at[l])

    def start_gather(k, p):
        s0 = k * CS
        g = gi[p]
        i0 = idx_v[0, pl.ds(s0, 16)]
        i1 = idx_v[1, pl.ds(s0, 16)]
        i2 = idx_v[2, pl.ds(s0, 16)]
        i3 = idx_v[3, pl.ds(s0, 16)]
        g[pl.ds(0, 16)] = i0
        g[pl.ds(16, 16)] = jnp.where(i1 != i0, i1, V)
        g[pl.ds(32, 16)] = jnp.where((i2 != i0) & (i2 != i1), i2, V)
        g[pl.ds(48, 16)] = jnp.where((i3 != i0) & (i3 != i1) & (i3 != i2),
                                     i3, V)
        pltpu.async_copy(tbl_hbm.at[g], rows[p], gsem[p])

    def wait_gather(p):
        pltpu.make_async_copy(tbl_hbm.at[gi[p]], rows[p], gsem[p]).wait()

    def reduce_chunk(p):
        r, s = rows[p], sums[p]

        def body(i, carry):
            for dblk in range(D // 16):
                sl = pl.ds(dblk * 16, 16)
                s[i, sl] = (r[i, sl] + r[CS + i, sl]
                            + r[2 * CS + i, sl] + r[3 * CS + i, sl])
            return carry

        lax.fori_loop(0, CS, body, 0)

    def start_wb(k, p):
        pltpu.async_copy(sums[p], out_hbm.at[pl.ds(base + k * CS, CS)],
                         wsem[p])

    def wait_wb(p):
        pltpu.make_async_copy(sums[p], out_hbm.at[pl.ds(0, CS)],
                              wsem[p]).wait()

    def chunk(k, p, wait_prev_wb, gather_ahead):
        wait_gather(p)
        if wait_prev_wb:
            wait_wb(p)
        reduce_chunk(p)
        start_wb(k, p)
        if gather_ahead:
            start_gather(k + 2, p)

    start_gather(0, 0)
    start_gather(1, 1)
    chunk(0, 0, False, True)
    chunk(1, 1, False, True)

    def pair(i, carry):
        k = 2 + 2 * i
        chunk(k, 0, True, True)
        chunk(k + 1, 1, True, True)
        return carry

    lax.fori_loop(0, (NCHUNK - 4) // 2, pair, 0)
    chunk(NCHUNK - 2, 0, True, False)
    chunk(NCHUNK - 1, 1, True, False)
    wait_wb(0)
    wait_wb(1)


def _tc_ln_body(s0_ref, s1_ref, s2_ref, b_ref, g_ref, be_ref, o_ref):
    acc = None
    for t, s_ref in enumerate((s0_ref, s1_ref, s2_ref)):
        x = s_ref[...] + b_ref[t][None, :]
        m = jnp.mean(x, axis=-1, keepdims=True)
        xc = x - m
        v = jnp.mean(xc * xc, axis=-1, keepdims=True)
        y = xc * lax.rsqrt(v + EPS) * g_ref[t][None, :] + be_ref[t][None, :]
        acc = y if acc is None else acc + y
    o_ref[...] = acc * (1.0 / 3.0)


_BB = 512


def _tc_ln(sums0, sums1, sums2, bias, gain, beta):
    bspec = pl.BlockSpec((_BB, D), lambda i: (i, 0))
    pspec = pl.BlockSpec((3, D), lambda i: (0, 0))
    return pl.pallas_call(
        _tc_ln_body,
        grid=(B // _BB,),
        in_specs=[bspec, bspec, bspec, pspec, pspec, pspec],
        out_specs=bspec,
        out_shape=jax.ShapeDtypeStruct((B, D), jnp.float32),
    )(sums0, sums1, sums2, bias, gain, beta)


def kernel(organs_idx, cells_idx, subcells_idx,
           W_o, b_o, g_o, be_o,
           W_c, b_c, g_c, be_c,
           W_s, b_s, g_s, be_s):
    sums = []
    for idx, W in ((organs_idx, W_o), (cells_idx, W_c), (subcells_idx, W_s)):
        tbl = _tc_transpose(W)
        sums.append(_sc_gather_sum(idx.T, tbl))
    bias = jnp.stack([b_o, b_c, b_s])
    gain = jnp.stack([g_o, g_c, g_s])
    beta = jnp.stack([be_o, be_c, be_s])
    return _tc_ln(sums[0], sums[1], sums[2], bias, gain, beta)


# speedup vs baseline: 43.1594x; 1.0658x over previous
import functools

import jax
import jax.numpy as jnp
from jax import lax
from jax.experimental import pallas as pl
from jax.experimental.pallas import tpu as pltpu
from jax.experimental.pallas import tpu_sc as plsc

B = 4096
L = 4
V = 8192
D = 512
EPS = 1e-5

NW = 32
PW = B // NW
CS = 16
NCHUNK = PW // CS
ROWS = CS * L

VPAD = V + 512

_mesh = plsc.VectorSubcoreMesh(core_axis_name="c", subcore_axis_name="s")


def _tc_transpose_body(w_ref, o_ref):
    i = pl.program_id(0)

    @pl.when(i < V // 512)
    def _():
        o_ref[...] = w_ref[...].T

    @pl.when(i == V // 512)
    def _():
        o_ref[...] = jnp.zeros_like(o_ref)


def _tc_transpose(w):
    return pl.pallas_call(
        _tc_transpose_body,
        grid=(V // 512 + 1,),
        in_specs=[pl.BlockSpec((D, 512), lambda i: (0, jnp.minimum(i, V // 512 - 1)))],
        out_specs=pl.BlockSpec((512, D), lambda i: (i, 0)),
        out_shape=jax.ShapeDtypeStruct((VPAD, D), jnp.float32),
    )(w)


_SC_SCRATCH = [
    pltpu.VMEM((2, L, PW), jnp.int32),
    pltpu.VMEM((ROWS,), jnp.int32),
    pltpu.VMEM((ROWS,), jnp.int32),
    pltpu.VMEM((ROWS, D), jnp.float32),
    pltpu.VMEM((ROWS, D), jnp.float32),
    pltpu.VMEM((CS, D), jnp.float32),
    pltpu.VMEM((CS, D), jnp.float32),
    pltpu.SemaphoreType.DMA,
    pltpu.SemaphoreType.DMA,
    pltpu.SemaphoreType.DMA,
    pltpu.SemaphoreType.DMA,
]


def _sc_impl(nlev, idx_hbm, tbls, out_hbm,
             idx_v, gi0, gi1, rows0, rows1, sums0, sums1,
             gs0, gs1, ws0, ws1):
    wid = lax.axis_index("s") * 2 + lax.axis_index("c")
    base = wid * PW
    gi = (gi0, gi1)
    rows = (rows0, rows1)
    sums = (sums0, sums1)
    gsem = (gs0, gs1)
    wsem = (ws0, ws1)

    for t in range(nlev):
        for l in range(L):
            pltpu.sync_copy(idx_hbm.at[t, l, pl.ds(base, PW)], idx_v.at[t, l])

    def start_gather(t, k, p):
        s0 = (k - t * NCHUNK) * CS
        g = gi[p]
        i0 = idx_v[t, 0, pl.ds(s0, 16)]
        i1 = idx_v[t, 1, pl.ds(s0, 16)]
        i2 = idx_v[t, 2, pl.ds(s0, 16)]
        i3 = idx_v[t, 3, pl.ds(s0, 16)]
        g[pl.ds(0, 16)] = i0
        g[pl.ds(16, 16)] = jnp.where(i1 != i0, i1, V)
        g[pl.ds(32, 16)] = jnp.where((i2 != i0) & (i2 != i1), i2, V)
        g[pl.ds(48, 16)] = jnp.where((i3 != i0) & (i3 != i1) & (i3 != i2),
                                     i3, V)
        pltpu.async_copy(tbls[t].at[g], rows[p], gsem[p])

    def reduce_chunk(p):
        r, s = rows[p], sums[p]

        def body(i, carry):
            for dblk in range(D // 16):
                sl = pl.ds(dblk * 16, 16)
                s[i, sl] = (r[i, sl] + r[CS + i, sl]
                            + r[2 * CS + i, sl] + r[3 * CS + i, sl])
            return carry

        lax.fori_loop(0, CS, body, 0)

    def wait_wb(p):
        pltpu.make_async_copy(sums[p], out_hbm.at[pl.ds(0, CS)],
                              wsem[p]).wait()

    def chunk(k, p, wait_prev_wb, tw, ta):
        pltpu.make_async_copy(tbls[tw].at[gi[p]], rows[p], gsem[p]).wait()
        if wait_prev_wb:
            wait_wb(p)
        reduce_chunk(p)
        q = tw * B + base + (k - tw * NCHUNK) * CS
        pltpu.async_copy(sums[p], out_hbm.at[pl.ds(q, CS)], wsem[p])
        if ta is not None:
            start_gather(ta, k + 2, p)

    start_gather(0, 0, 0)
    start_gather(0, 1, 1)
    chunk(0, 0, False, 0, 0)
    chunk(1, 1, False, 0, 0)
    for t in range(nlev):
        kb = t * NCHUNK
        lo = 2 if t == 0 else kb
        hi = kb + NCHUNK - 2
        npairs = (hi - lo) // 2

        def pair(i, carry, lo=lo, t=t):
            k = lo + 2 * i
            chunk(k, 0, True, t, t)
            chunk(k + 1, 1, True, t, t)
            return carry

        lax.fori_loop(0, npairs, pair, 0)
        ta = t + 1 if t + 1 < nlev else None
        chunk(kb + NCHUNK - 2, 0, True, t, ta)
        chunk(kb + NCHUNK - 1, 1, True, t, ta)
    wait_wb(0)
    wait_wb(1)


@functools.partial(
    pl.kernel,
    mesh=_mesh,
    out_type=jax.ShapeDtypeStruct((B, D), jnp.float32),
    scratch_types=_SC_SCRATCH,
)
def _sc_gather_sum1(idx_hbm, tbl_hbm, out_hbm, *scratch):
    _sc_impl(1, idx_hbm, (tbl_hbm,), out_hbm, *scratch)


@functools.partial(
    pl.kernel,
    mesh=_mesh,
    out_type=jax.ShapeDtypeStruct((2 * B, D), jnp.float32),
    scratch_types=_SC_SCRATCH,
)
def _sc_gather_sum2(idx_hbm, tbl1_hbm, tbl2_hbm, out_hbm, *scratch):
    _sc_impl(2, idx_hbm, (tbl1_hbm, tbl2_hbm), out_hbm, *scratch)


def _ln(x, g, be):
    m = jnp.mean(x, axis=-1, keepdims=True)
    xc = x - m
    v = jnp.mean(xc * xc, axis=-1, keepdims=True)
    return xc * lax.rsqrt(v + EPS) * g + be


def _tc_ln0_body(s_ref, b_ref, g_ref, be_ref, o_ref):
    o_ref[...] = _ln(s_ref[...] + b_ref[...], g_ref[...], be_ref[...])


def _tc_fin_body(y0_ref, s1_ref, s2_ref, b_ref, g_ref, be_ref, o_ref):
    y1 = _ln(s1_ref[...] + b_ref[0][None, :], g_ref[0][None, :],
             be_ref[0][None, :])
    y2 = _ln(s2_ref[...] + b_ref[1][None, :], g_ref[1][None, :],
             be_ref[1][None, :])
    o_ref[...] = (y0_ref[...] + y1 + y2) * (1.0 / 3.0)


_BB = 512


def kernel(organs_idx, cells_idx, subcells_idx,
           W_o, b_o, g_o, be_o,
           W_c, b_c, g_c, be_c,
           W_s, b_s, g_s, be_s):
    t0 = _tc_transpose(W_o)
    s0 = _sc_gather_sum1(organs_idx.T[None], t0)
    t1 = _tc_transpose(W_c)
    t2 = _tc_transpose(W_s)
    s12 = _sc_gather_sum2(jnp.stack([cells_idx.T, subcells_idx.T]), t1, t2)

    bspec = pl.BlockSpec((_BB, D), lambda i: (i, 0))
    vspec = pl.BlockSpec((1, D), lambda i: (0, 0))
    y0 = pl.pallas_call(
        _tc_ln0_body,
        grid=(B // _BB,),
        in_specs=[bspec, vspec, vspec, vspec],
        out_specs=bspec,
        out_shape=jax.ShapeDtypeStruct((B, D), jnp.float32),
    )(s0, b_o[None], g_o[None], be_o[None])

    pspec = pl.BlockSpec((2, D), lambda i: (0, 0))
    s1spec = pl.BlockSpec((_BB, D), lambda i: (i, 0))
    s2spec = pl.BlockSpec((_BB, D), lambda i: (B // _BB + i, 0))
    return pl.pallas_call(
        _tc_fin_body,
        grid=(B // _BB,),
        in_specs=[bspec, s1spec, s2spec, pspec, pspec, pspec],
        out_specs=bspec,
        out_shape=jax.ShapeDtypeStruct((B, D), jnp.float32),
    )(y0, s12, s12,
      jnp.stack([b_c, b_s]), jnp.stack([g_c, g_s]), jnp.stack([be_c, be_s]))


# speedup vs baseline: 44.6844x; 1.0353x over previous
import functools

import jax
import jax.numpy as jnp
from jax import lax
from jax.experimental import pallas as pl
from jax.experimental.pallas import tpu as pltpu
from jax.experimental.pallas import tpu_sc as plsc

B = 4096
L = 4
V = 8192
D = 512
EPS = 1e-5

NW = 32
PW = B // NW
CS = 16
NCHUNK = PW // CS
ROWS = CS * L

VPAD = V + 1024

_mesh = plsc.VectorSubcoreMesh(core_axis_name="c", subcore_axis_name="s")


_TB = 1024


def _tc_transpose_body(w_ref, o_ref):
    i = pl.program_id(0)

    @pl.when(i < V // _TB)
    def _():
        o_ref[...] = w_ref[...].T

    @pl.when(i == V // _TB)
    def _():
        o_ref[...] = jnp.zeros_like(o_ref)


def _tc_transpose(w):
    return pl.pallas_call(
        _tc_transpose_body,
        grid=(V // _TB + 1,),
        in_specs=[pl.BlockSpec((D, _TB), lambda i: (0, jnp.minimum(i, V // _TB - 1)))],
        out_specs=pl.BlockSpec((_TB, D), lambda i: (i, 0)),
        out_shape=jax.ShapeDtypeStruct((VPAD, D), jnp.float32),
    )(w)


_SC_SCRATCH = [
    pltpu.VMEM((2, L, PW), jnp.int32),
    pltpu.VMEM((ROWS,), jnp.int32),
    pltpu.VMEM((ROWS,), jnp.int32),
    pltpu.VMEM((ROWS, D), jnp.float32),
    pltpu.VMEM((ROWS, D), jnp.float32),
    pltpu.VMEM((CS, D), jnp.float32),
    pltpu.VMEM((CS, D), jnp.float32),
    pltpu.SemaphoreType.DMA,
    pltpu.SemaphoreType.DMA,
    pltpu.SemaphoreType.DMA,
    pltpu.SemaphoreType.DMA,
]


def _sc_impl(nlev, idx_hbm, tbls, out_hbm,
             idx_v, gi0, gi1, rows0, rows1, sums0, sums1,
             gs0, gs1, ws0, ws1):
    wid = lax.axis_index("s") * 2 + lax.axis_index("c")
    base = wid * PW
    gi = (gi0, gi1)
    rows = (rows0, rows1)
    sums = (sums0, sums1)
    gsem = (gs0, gs1)
    wsem = (ws0, ws1)

    for t in range(nlev):
        for l in range(L):
            pltpu.sync_copy(idx_hbm.at[t, l, pl.ds(base, PW)], idx_v.at[t, l])

    def start_gather(t, k, p):
        s0 = (k - t * NCHUNK) * CS
        g = gi[p]
        i0 = idx_v[t, 0, pl.ds(s0, 16)]
        i1 = idx_v[t, 1, pl.ds(s0, 16)]
        i2 = idx_v[t, 2, pl.ds(s0, 16)]
        i3 = idx_v[t, 3, pl.ds(s0, 16)]
        g[pl.ds(0, 16)] = i0
        g[pl.ds(16, 16)] = jnp.where(i1 != i0, i1, V)
        g[pl.ds(32, 16)] = jnp.where((i2 != i0) & (i2 != i1), i2, V)
        g[pl.ds(48, 16)] = jnp.where((i3 != i0) & (i3 != i1) & (i3 != i2),
                                     i3, V)
        pltpu.async_copy(tbls[t].at[g], rows[p], gsem[p])

    def reduce_chunk(p):
        r, s = rows[p], sums[p]

        def body(i, carry):
            for dblk in range(D // 16):
                sl = pl.ds(dblk * 16, 16)
                s[i, sl] = (r[i, sl] + r[CS + i, sl]
                            + r[2 * CS + i, sl] + r[3 * CS + i, sl])
            return carry

        lax.fori_loop(0, CS, body, 0)

    def wait_wb(p):
        pltpu.make_async_copy(sums[p], out_hbm.at[pl.ds(0, CS)],
                              wsem[p]).wait()

    def chunk(k, p, wait_prev_wb, tw, ta):
        pltpu.make_async_copy(tbls[tw].at[gi[p]], rows[p], gsem[p]).wait()
        if wait_prev_wb:
            wait_wb(p)
        reduce_chunk(p)
        q = tw * B + base + (k - tw * NCHUNK) * CS
        pltpu.async_copy(sums[p], out_hbm.at[pl.ds(q, CS)], wsem[p])
        if ta is not None:
            start_gather(ta, k + 2, p)

    start_gather(0, 0, 0)
    start_gather(0, 1, 1)
    chunk(0, 0, False, 0, 0)
    chunk(1, 1, False, 0, 0)
    for t in range(nlev):
        kb = t * NCHUNK
        lo = 2 if t == 0 else kb
        hi = kb + NCHUNK - 2
        npairs = (hi - lo) // 2

        def pair(i, carry, lo=lo, t=t):
            k = lo + 2 * i
            chunk(k, 0, True, t, t)
            chunk(k + 1, 1, True, t, t)
            return carry

        lax.fori_loop(0, npairs, pair, 0)
        ta = t + 1 if t + 1 < nlev else None
        chunk(kb + NCHUNK - 2, 0, True, t, ta)
        chunk(kb + NCHUNK - 1, 1, True, t, ta)
    wait_wb(0)
    wait_wb(1)


@functools.partial(
    pl.kernel,
    mesh=_mesh,
    out_type=jax.ShapeDtypeStruct((B, D), jnp.float32),
    scratch_types=_SC_SCRATCH,
)
def _sc_gather_sum1(idx_hbm, tbl_hbm, out_hbm, *scratch):
    _sc_impl(1, idx_hbm, (tbl_hbm,), out_hbm, *scratch)


@functools.partial(
    pl.kernel,
    mesh=_mesh,
    out_type=jax.ShapeDtypeStruct((2 * B, D), jnp.float32),
    scratch_types=_SC_SCRATCH,
)
def _sc_gather_sum2(idx_hbm, tbl1_hbm, tbl2_hbm, out_hbm, *scratch):
    _sc_impl(2, idx_hbm, (tbl1_hbm, tbl2_hbm), out_hbm, *scratch)


def _ln(x, g, be):
    m = jnp.mean(x, axis=-1, keepdims=True)
    xc = x - m
    v = jnp.mean(xc * xc, axis=-1, keepdims=True)
    return xc * lax.rsqrt(v + EPS) * g + be


def _tc_ln0_body(s_ref, b_ref, g_ref, be_ref, o_ref):
    o_ref[...] = _ln(s_ref[...] + b_ref[...], g_ref[...], be_ref[...])


def _tc_fin_body(y0_ref, s1_ref, s2_ref, b_ref, g_ref, be_ref, o_ref):
    y1 = _ln(s1_ref[...] + b_ref[0][None, :], g_ref[0][None, :],
             be_ref[0][None, :])
    y2 = _ln(s2_ref[...] + b_ref[1][None, :], g_ref[1][None, :],
             be_ref[1][None, :])
    o_ref[...] = (y0_ref[...] + y1 + y2) * (1.0 / 3.0)


_BB = 512


def kernel(organs_idx, cells_idx, subcells_idx,
           W_o, b_o, g_o, be_o,
           W_c, b_c, g_c, be_c,
           W_s, b_s, g_s, be_s):
    t0 = _tc_transpose(W_o)
    s0 = _sc_gather_sum1(organs_idx.T[None], t0)
    t1 = _tc_transpose(W_c)
    t2 = _tc_transpose(W_s)
    s12 = _sc_gather_sum2(jnp.stack([cells_idx.T, subcells_idx.T]), t1, t2)

    bspec = pl.BlockSpec((_BB, D), lambda i: (i, 0))
    vspec = pl.BlockSpec((1, D), lambda i: (0, 0))
    y0 = pl.pallas_call(
        _tc_ln0_body,
        grid=(B // _BB,),
        in_specs=[bspec, vspec, vspec, vspec],
        out_specs=bspec,
        out_shape=jax.ShapeDtypeStruct((B, D), jnp.float32),
    )(s0, b_o[None], g_o[None], be_o[None])

    pspec = pl.BlockSpec((2, D), lambda i: (0, 0))
    s1spec = pl.BlockSpec((_BB, D), lambda i: (i, 0))
    s2spec = pl.BlockSpec((_BB, D), lambda i: (B // _BB + i, 0))
    return pl.pallas_call(
        _tc_fin_body,
        grid=(B // _BB,),
        in_specs=[bspec, s1spec, s2spec, pspec, pspec, pspec],
        out_specs=bspec,
        out_shape=jax.ShapeDtypeStruct((B, D), jnp.float32),
    )(y0, s12, s12,
      jnp.stack([b_c, b_s]), jnp.stack([g_c, g_s]), jnp.stack([be_c, be_s]))


# speedup vs baseline: 45.7608x; 1.0241x over previous
import functools

import jax
import jax.numpy as jnp
from jax import lax
from jax.experimental import pallas as pl
from jax.experimental.pallas import tpu as pltpu
from jax.experimental.pallas import tpu_sc as plsc

B = 4096
L = 4
V = 8192
D = 512
EPS = 1e-5

NW = 32
PW = B // NW
CS = 16
NCHUNK = PW // CS
ROWS = CS * L

VPAD = V + 2048

_mesh = plsc.VectorSubcoreMesh(core_axis_name="c", subcore_axis_name="s")


_TB = 2048


def _tc_transpose_body(w_ref, o_ref):
    i = pl.program_id(0)

    @pl.when(i < V // _TB)
    def _():
        o_ref[...] = w_ref[...].T

    @pl.when(i == V // _TB)
    def _():
        o_ref[...] = jnp.zeros_like(o_ref)


def _tc_transpose(w):
    return pl.pallas_call(
        _tc_transpose_body,
        grid=(V // _TB + 1,),
        in_specs=[pl.BlockSpec((D, _TB), lambda i: (0, jnp.minimum(i, V // _TB - 1)))],
        out_specs=pl.BlockSpec((_TB, D), lambda i: (i, 0)),
        out_shape=jax.ShapeDtypeStruct((VPAD, D), jnp.float32),
    )(w)


_SC_SCRATCH = [
    pltpu.VMEM((2, L, PW), jnp.int32),
    pltpu.VMEM((ROWS,), jnp.int32),
    pltpu.VMEM((ROWS,), jnp.int32),
    pltpu.VMEM((ROWS, D), jnp.float32),
    pltpu.VMEM((ROWS, D), jnp.float32),
    pltpu.VMEM((CS, D), jnp.float32),
    pltpu.VMEM((CS, D), jnp.float32),
    pltpu.SemaphoreType.DMA,
    pltpu.SemaphoreType.DMA,
    pltpu.SemaphoreType.DMA,
    pltpu.SemaphoreType.DMA,
]


def _sc_impl(nlev, idx_hbm, tbls, out_hbm,
             idx_v, gi0, gi1, rows0, rows1, sums0, sums1,
             gs0, gs1, ws0, ws1):
    wid = lax.axis_index("s") * 2 + lax.axis_index("c")
    base = wid * PW
    gi = (gi0, gi1)
    rows = (rows0, rows1)
    sums = (sums0, sums1)
    gsem = (gs0, gs1)
    wsem = (ws0, ws1)

    pltpu.sync_copy(idx_hbm.at[:, :, pl.ds(base, PW)],
                    idx_v.at[pl.ds(0, nlev)])

    def start_gather(t, k, p):
        s0 = (k - t * NCHUNK) * CS
        g = gi[p]
        i0 = idx_v[t, 0, pl.ds(s0, 16)]
        i1 = idx_v[t, 1, pl.ds(s0, 16)]
        i2 = idx_v[t, 2, pl.ds(s0, 16)]
        i3 = idx_v[t, 3, pl.ds(s0, 16)]
        g[pl.ds(0, 16)] = i0
        g[pl.ds(16, 16)] = jnp.where(i1 != i0, i1, V)
        g[pl.ds(32, 16)] = jnp.where((i2 != i0) & (i2 != i1), i2, V)
        g[pl.ds(48, 16)] = jnp.where((i3 != i0) & (i3 != i1) & (i3 != i2),
                                     i3, V)
        pltpu.async_copy(tbls[t].at[g], rows[p], gsem[p])

    def reduce_chunk(p):
        r, s = rows[p], sums[p]

        def body(i, carry):
            for dblk in range(D // 16):
                sl = pl.ds(dblk * 16, 16)
                s[i, sl] = (r[i, sl] + r[CS + i, sl]
                            + r[2 * CS + i, sl] + r[3 * CS + i, sl])
            return carry

        lax.fori_loop(0, CS, body, 0)

    def wait_wb(p):
        pltpu.make_async_copy(sums[p], out_hbm.at[pl.ds(0, CS)],
                              wsem[p]).wait()

    def chunk(k, p, wait_prev_wb, tw, ta):
        pltpu.make_async_copy(tbls[tw].at[gi[p]], rows[p], gsem[p]).wait()
        if wait_prev_wb:
            wait_wb(p)
        reduce_chunk(p)
        q = tw * B + base + (k - tw * NCHUNK) * CS
        pltpu.async_copy(sums[p], out_hbm.at[pl.ds(q, CS)], wsem[p])
        if ta is not None:
            start_gather(ta, k + 2, p)

    start_gather(0, 0, 0)
    start_gather(0, 1, 1)
    chunk(0, 0, False, 0, 0)
    chunk(1, 1, False, 0, 0)
    for t in range(nlev):
        kb = t * NCHUNK
        lo = 2 if t == 0 else kb
        hi = kb + NCHUNK - 2
        npairs = (hi - lo) // 2

        def pair(i, carry, lo=lo, t=t):
            k = lo + 2 * i
            chunk(k, 0, True, t, t)
            chunk(k + 1, 1, True, t, t)
            return carry

        lax.fori_loop(0, npairs, pair, 0)
        ta = t + 1 if t + 1 < nlev else None
        chunk(kb + NCHUNK - 2, 0, True, t, ta)
        chunk(kb + NCHUNK - 1, 1, True, t, ta)
    wait_wb(0)
    wait_wb(1)


@functools.partial(
    pl.kernel,
    mesh=_mesh,
    out_type=jax.ShapeDtypeStruct((B, D), jnp.float32),
    scratch_types=_SC_SCRATCH,
)
def _sc_gather_sum1(idx_hbm, tbl_hbm, out_hbm, *scratch):
    _sc_impl(1, idx_hbm, (tbl_hbm,), out_hbm, *scratch)


@functools.partial(
    pl.kernel,
    mesh=_mesh,
    out_type=jax.ShapeDtypeStruct((2 * B, D), jnp.float32),
    scratch_types=_SC_SCRATCH,
)
def _sc_gather_sum2(idx_hbm, tbl1_hbm, tbl2_hbm, out_hbm, *scratch):
    _sc_impl(2, idx_hbm, (tbl1_hbm, tbl2_hbm), out_hbm, *scratch)


def _ln(x, g, be):
    m = jnp.mean(x, axis=-1, keepdims=True)
    xc = x - m
    v = jnp.mean(xc * xc, axis=-1, keepdims=True)
    return xc * lax.rsqrt(v + EPS) * g + be


def _tc_ln0_body(s_ref, b_ref, g_ref, be_ref, o_ref):
    o_ref[...] = _ln(s_ref[...] + b_ref[...], g_ref[...], be_ref[...])


def _tc_fin_body(y0_ref, s1_ref, s2_ref, b_ref, g_ref, be_ref, o_ref):
    y1 = _ln(s1_ref[...] + b_ref[0][None, :], g_ref[0][None, :],
             be_ref[0][None, :])
    y2 = _ln(s2_ref[...] + b_ref[1][None, :], g_ref[1][None, :],
             be_ref[1][None, :])
    o_ref[...] = (y0_ref[...] + y1 + y2) * (1.0 / 3.0)


_BB = 512


def kernel(organs_idx, cells_idx, subcells_idx,
           W_o, b_o, g_o, be_o,
           W_c, b_c, g_c, be_c,
           W_s, b_s, g_s, be_s):
    t0 = _tc_transpose(W_o)
    s0 = _sc_gather_sum1(organs_idx.T[None], t0)
    t1 = _tc_transpose(W_c)
    t2 = _tc_transpose(W_s)
    s12 = _sc_gather_sum2(jnp.stack([cells_idx.T, subcells_idx.T]), t1, t2)

    bspec = pl.BlockSpec((_BB, D), lambda i: (i, 0))
    vspec = pl.BlockSpec((1, D), lambda i: (0, 0))
    y0 = pl.pallas_call(
        _tc_ln0_body,
        grid=(B // _BB,),
        in_specs=[bspec, vspec, vspec, vspec],
        out_specs=bspec,
        out_shape=jax.ShapeDtypeStruct((B, D), jnp.float32),
    )(s0, b_o[None], g_o[None], be_o[None])

    pspec = pl.BlockSpec((2, D), lambda i: (0, 0))
    s1spec = pl.BlockSpec((_BB, D), lambda i: (i, 0))
    s2spec = pl.BlockSpec((_BB, D), lambda i: (B // _BB + i, 0))
    return pl.pallas_call(
        _tc_fin_body,
        grid=(B // _BB,),
        in_specs=[bspec, s1spec, s2spec, pspec, pspec, pspec],
        out_specs=bspec,
        out_shape=jax.ShapeDtypeStruct((B, D), jnp.float32),
    )(y0, s12, s12,
      jnp.stack([b_c, b_s]), jnp.stack([g_c, g_s]), jnp.stack([be_c, be_s]))


# speedup vs baseline: 46.0689x; 1.0067x over previous
import functools

import jax
import jax.numpy as jnp
from jax import lax
from jax.experimental import pallas as pl
from jax.experimental.pallas import tpu as pltpu
from jax.experimental.pallas import tpu_sc as plsc

B = 4096
L = 4
V = 8192
D = 512
EPS = 1e-5

NW = 32
PW = B // NW
CS = 16
NCHUNK = PW // CS
ROWS = CS * L

VPAD = V + 2048

_mesh = plsc.VectorSubcoreMesh(core_axis_name="c", subcore_axis_name="s")


_TB = 2048


def _tc_transpose_body(w_ref, o_ref):
    i = pl.program_id(0)

    @pl.when(i < V // _TB)
    def _():
        o_ref[...] = w_ref[...].T

    @pl.when(i == V // _TB)
    def _():
        o_ref[...] = jnp.zeros_like(o_ref)


def _tc_transpose(w):
    return pl.pallas_call(
        _tc_transpose_body,
        grid=(V // _TB + 1,),
        in_specs=[pl.BlockSpec((D, _TB), lambda i: (0, jnp.minimum(i, V // _TB - 1)))],
        out_specs=pl.BlockSpec((_TB, D), lambda i: (i, 0)),
        out_shape=jax.ShapeDtypeStruct((VPAD, D), jnp.float32),
    )(w)


_SC_SCRATCH = [
    pltpu.VMEM((2, L, PW), jnp.int32),
    pltpu.VMEM((ROWS,), jnp.int32),
    pltpu.VMEM((ROWS,), jnp.int32),
    pltpu.VMEM((ROWS, D), jnp.float32),
    pltpu.VMEM((ROWS, D), jnp.float32),
    pltpu.VMEM((CS, D), jnp.float32),
    pltpu.VMEM((CS, D), jnp.float32),
    pltpu.SemaphoreType.DMA,
    pltpu.SemaphoreType.DMA,
    pltpu.SemaphoreType.DMA,
    pltpu.SemaphoreType.DMA,
]


def _sc_impl(nlev, idx_hbm, tbls, out_hbm,
             idx_v, gi0, gi1, rows0, rows1, sums0, sums1,
             gs0, gs1, ws0, ws1):
    wid = lax.axis_index("s") * 2 + lax.axis_index("c")
    base = wid * PW
    gi = (gi0, gi1)
    rows = (rows0, rows1)
    sums = (sums0, sums1)
    gsem = (gs0, gs1)
    wsem = (ws0, ws1)

    pltpu.sync_copy(idx_hbm.at[:, :, pl.ds(base, PW)],
                    idx_v.at[pl.ds(0, nlev)])

    def start_gather(t, k, p):
        s0 = (k - t * NCHUNK) * CS
        g = gi[p]
        i0 = idx_v[t, 0, pl.ds(s0, 16)]
        i1 = idx_v[t, 1, pl.ds(s0, 16)]
        i2 = idx_v[t, 2, pl.ds(s0, 16)]
        i3 = idx_v[t, 3, pl.ds(s0, 16)]
        g[pl.ds(0, 16)] = i0
        g[pl.ds(16, 16)] = jnp.where(i1 != i0, i1, V)
        g[pl.ds(32, 16)] = jnp.where((i2 != i0) & (i2 != i1), i2, V)
        g[pl.ds(48, 16)] = jnp.where((i3 != i0) & (i3 != i1) & (i3 != i2),
                                     i3, V)
        pltpu.async_copy(tbls[t].at[g], rows[p], gsem[p])

    def reduce_chunk(p):
        r, s = rows[p], sums[p]

        def body(i, carry):
            for dblk in range(D // 16):
                sl = pl.ds(dblk * 16, 16)
                s[i, sl] = (r[i, sl] + r[CS + i, sl]
                            + r[2 * CS + i, sl] + r[3 * CS + i, sl])
            return carry

        lax.fori_loop(0, CS, body, 0)

    def wait_wb(p):
        pltpu.make_async_copy(sums[p], out_hbm.at[pl.ds(0, CS)],
                              wsem[p]).wait()

    def chunk(k, p, wait_prev_wb, tw, ta):
        pltpu.make_async_copy(tbls[tw].at[gi[p]], rows[p], gsem[p]).wait()
        if wait_prev_wb:
            wait_wb(p)
        reduce_chunk(p)
        q = tw * B + base + (k - tw * NCHUNK) * CS
        pltpu.async_copy(sums[p], out_hbm.at[pl.ds(q, CS)], wsem[p])
        if ta is not None:
            start_gather(ta, k + 2, p)

    start_gather(0, 0, 0)
    start_gather(0, 1, 1)
    chunk(0, 0, False, 0, 0)
    chunk(1, 1, False, 0, 0)
    for t in range(nlev):
        kb = t * NCHUNK
        lo = 2 if t == 0 else kb
        hi = kb + NCHUNK - 2
        npairs = (hi - lo) // 2

        def pair(i, carry, lo=lo, t=t):
            k = lo + 2 * i
            chunk(k, 0, True, t, t)
            chunk(k + 1, 1, True, t, t)
            return carry

        lax.fori_loop(0, npairs, pair, 0)
        ta = t + 1 if t + 1 < nlev else None
        chunk(kb + NCHUNK - 2, 0, True, t, ta)
        chunk(kb + NCHUNK - 1, 1, True, t, ta)
    wait_wb(0)
    wait_wb(1)


@functools.partial(
    pl.kernel,
    mesh=_mesh,
    out_type=jax.ShapeDtypeStruct((B, D), jnp.float32),
    scratch_types=_SC_SCRATCH,
)
def _sc_gather_sum1(idx_hbm, tbl_hbm, out_hbm, *scratch):
    _sc_impl(1, idx_hbm, (tbl_hbm,), out_hbm, *scratch)


@functools.partial(
    pl.kernel,
    mesh=_mesh,
    out_type=jax.ShapeDtypeStruct((2 * B, D), jnp.float32),
    scratch_types=_SC_SCRATCH,
)
def _sc_gather_sum2(idx_hbm, tbl1_hbm, tbl2_hbm, out_hbm, *scratch):
    _sc_impl(2, idx_hbm, (tbl1_hbm, tbl2_hbm), out_hbm, *scratch)


def _ln(x, g, be):
    m = jnp.mean(x, axis=-1, keepdims=True)
    xc = x - m
    v = jnp.mean(xc * xc, axis=-1, keepdims=True)
    return xc * lax.rsqrt(v + EPS) * g + be


def _tc_ln0_body(s_ref, b_ref, g_ref, be_ref, o_ref):
    o_ref[...] = _ln(s_ref[...] + b_ref[...], g_ref[...], be_ref[...])


def _tc_fin_body(y0_ref, s1_ref, s2_ref, b_ref, g_ref, be_ref, o_ref):
    y1 = _ln(s1_ref[...] + b_ref[0][None, :], g_ref[0][None, :],
             be_ref[0][None, :])
    y2 = _ln(s2_ref[...] + b_ref[1][None, :], g_ref[1][None, :],
             be_ref[1][None, :])
    o_ref[...] = (y0_ref[...] + y1 + y2) * (1.0 / 3.0)


_BB = 1024


def kernel(organs_idx, cells_idx, subcells_idx,
           W_o, b_o, g_o, be_o,
           W_c, b_c, g_c, be_c,
           W_s, b_s, g_s, be_s):
    t0 = _tc_transpose(W_o)
    s0 = _sc_gather_sum1(organs_idx.T[None], t0)
    t1 = _tc_transpose(W_c)
    t2 = _tc_transpose(W_s)
    s12 = _sc_gather_sum2(jnp.stack([cells_idx.T, subcells_idx.T]), t1, t2)

    bspec = pl.BlockSpec((_BB, D), lambda i: (i, 0))
    vspec = pl.BlockSpec((1, D), lambda i: (0, 0))
    y0 = pl.pallas_call(
        _tc_ln0_body,
        grid=(B // _BB,),
        in_specs=[bspec, vspec, vspec, vspec],
        out_specs=bspec,
        out_shape=jax.ShapeDtypeStruct((B, D), jnp.float32),
    )(s0, b_o[None], g_o[None], be_o[None])

    pspec = pl.BlockSpec((2, D), lambda i: (0, 0))
    s1spec = pl.BlockSpec((_BB, D), lambda i: (i, 0))
    s2spec = pl.BlockSpec((_BB, D), lambda i: (B // _BB + i, 0))
    return pl.pallas_call(
        _tc_fin_body,
        grid=(B // _BB,),
        in_specs=[bspec, s1spec, s2spec, pspec, pspec, pspec],
        out_specs=bspec,
        out_shape=jax.ShapeDtypeStruct((B, D), jnp.float32),
    )(y0, s12, s12,
      jnp.stack([b_c, b_s]), jnp.stack([g_c, g_s]), jnp.stack([be_c, be_s]))
